# Initial kernel scaffold; baseline (speedup 1.0000x reference)
import jax
import jax.numpy as jnp
from jax.experimental import pallas as pl


def kernel(x, edge_index, batch, W1, b1, W2, b2):
    raise NotImplementedError("write your pallas kernel here")


# speedup vs baseline: 10.7554x; 10.7554x over previous
import functools

import jax
import jax.numpy as jnp
from jax import lax
from jax.experimental import pallas as pl
from jax.experimental.pallas import tpu as pltpu
from jax.experimental.pallas import tpu_sc as plsc

_N = 10000
_E = 160000
_D = 256
_H = 512
_NP = 10240
_EP = 163840
_NT = 16
_CH = 80
_RPT = _NP // _NT
_f32 = jnp.float32


def _agg1_sc(xlo, xhi, e3):
  mesh = plsc.VectorSubcoreMesh(core_axis_name="c", subcore_axis_name="s")

  @functools.partial(
      pl.kernel,
      mesh=mesh,
      out_type=[jax.ShapeDtypeStruct((_NP, 128), _f32),
                jax.ShapeDtypeStruct((_NP, 128), _f32)],
      scratch_types=[
          pltpu.VMEM((2, 2, 128), jnp.int32),
          pltpu.VMEM((128, 128), _f32),
          pltpu.VMEM((128, 128), _f32),
          pltpu.VMEM_SHARED((_NP, 128), _f32),
          pltpu.SemaphoreType.DMA,
          pltpu.SemaphoreType.DMA,
          pltpu.SemaphoreType.DMA,
          pltpu.SemaphoreType.DMA,
      ],
  )
  def k(xlo_h, xhi_h, e3_h, out0, out1,
        ix, b0, b1, acc, semi0, semi1, semg0, semg1):
    c = lax.axis_index("c")
    s = lax.axis_index("s")
    t0 = s * _CH

    def zb(i, carry):
      r = i // 8
      cc = lax.rem(i, 8)
      b0[r, pl.ds(cc * 16, 16)] = jnp.zeros((16,), _f32)
      return carry

    lax.fori_loop(0, 1024, zb, 0)
    rowbase = s * _RPT
    for kk in range(_RPT // 128):
      pltpu.sync_copy(b0, acc.at[pl.ds(rowbase + kk * 128, 128)])
    plsc.subcore_barrier()

    def run(x_h, out_h):
      def idxload(j, par, sem):
        return pltpu.make_async_copy(e3_h.at[t0 + j], ix.at[par], sem)

      def gather(par, buf, sem):
        return pltpu.make_async_copy(x_h.at[ix.at[par, 0]], buf, sem)

      idxload(0, 0, semi0).start()
      idxload(1, 1, semi1).start()
      idxload(0, 0, semi0).wait()
      gather(0, b0, semg0).start()
      idxload(1, 1, semi1).wait()
      gather(1, b1, semg1).start()

      def body(i, carry):
        j0 = i * 2
        more = i < _CH // 2 - 1
        gather(0, b0, semg0).wait()
        pltpu.sync_copy(b0, acc.at[ix.at[0, 1]], add=True)

        @pl.when(more)
        def _():
          idxload(j0 + 2, 0, semi0).start()

        gather(1, b1, semg1).wait()
        pltpu.sync_copy(b1, acc.at[ix.at[1, 1]], add=True)

        @pl.when(more)
        def _():
          idxload(j0 + 3, 1, semi1).start()
          idxload(0, 0, semi0).wait()
          gather(0, b0, semg0).start()
          idxload(0, 1, semi1).wait()
          gather(1, b1, semg1).start()

        return carry

      lax.fori_loop(0, _CH // 2, body, 0)
      plsc.subcore_barrier()
      pltpu.sync_copy(acc.at[pl.ds(rowbase, _RPT)],
                      out_h.at[pl.ds(rowbase, _RPT)])

    @pl.when(c == 0)
    def _():
      run(xlo_h, out0)

    @pl.when(c == 1)
    def _():
      run(xhi_h, out1)

  return k(xlo, xhi, e3)


def _aggs_sc(s0p, src2d, dst2d):
  mesh = plsc.VectorSubcoreMesh(core_axis_name="c", subcore_axis_name="s")
  ch2 = _EP // 128 // 32

  @functools.partial(
      pl.kernel,
      mesh=mesh,
      out_type=jax.ShapeDtypeStruct((2, _NP), _f32),
      scratch_types=[
          pltpu.VMEM((ch2, 128), jnp.int32),
          pltpu.VMEM((ch2, 128), jnp.int32),
          pltpu.VMEM((128,), _f32),
          pltpu.VMEM_SHARED((_NP,), _f32),
          pltpu.VMEM_SHARED((_NP,), _f32),
      ],
  )
  def k(s_h, s2_h, d2_h, out, src_v, dst_v, vals_v, s_sh, acc):
    c = lax.axis_index("c")
    s = lax.axis_index("s")
    wid = s * 2 + c
    t0 = wid * ch2
    pltpu.sync_copy(s2_h.at[pl.ds(t0, ch2)], src_v)
    pltpu.sync_copy(d2_h.at[pl.ds(t0, ch2)], dst_v)
    for g in range(8):
      vals_v[pl.ds(g * 16, 16)] = jnp.zeros((16,), _f32)
    rowbase = s * _RPT
    for kk in range(_RPT // 128):
      pltpu.sync_copy(vals_v, acc.at[pl.ds(rowbase + kk * 128, 128)])

    @pl.when(s == 0)
    def _():
      pltpu.sync_copy(s_h, s_sh)

    plsc.subcore_barrier()

    def body(j, carry):
      pltpu.sync_copy(s_sh.at[src_v.at[j]], vals_v)
      pltpu.sync_copy(vals_v, acc.at[dst_v.at[j]], add=True)
      return carry

    lax.fori_loop(0, ch2, body, 0)
    plsc.subcore_barrier()
    pltpu.sync_copy(acc.at[pl.ds(rowbase, _RPT)],
                    out.at[c, pl.ds(rowbase, _RPT)])

  return k(s0p, src2d, dst2d)


def _mlp_tc(x, agg_lo, agg_hi, W1, b1r, W2):
  bn = 1000

  def body(x_r, alo_r, ahi_r, w1_r, b1_r, w2_r, s0_r):
    hi = jnp.dot(x_r[:, :128] + alo_r[...], w1_r[:128, :],
                 preferred_element_type=_f32,
                 precision=lax.Precision.HIGHEST)
    hi = hi + jnp.dot(x_r[:, 128:] + ahi_r[...], w1_r[128:, :],
                      preferred_element_type=_f32,
                      precision=lax.Precision.HIGHEST)
    h = jnp.maximum(hi + b1_r[...], 0.0)
    s0_r[...] = jnp.dot(h, w2_r[...], preferred_element_type=_f32,
                        precision=lax.Precision.HIGHEST)

  return pl.pallas_call(
      body,
      grid=(_N // bn,),
      in_specs=[
          pl.BlockSpec((bn, _D), lambda i: (i, 0)),
          pl.BlockSpec((bn, 128), lambda i: (i, 0)),
          pl.BlockSpec((bn, 128), lambda i: (i, 0)),
          pl.BlockSpec((_D, _H), lambda i: (0, 0)),
          pl.BlockSpec((1, _H), lambda i: (0, 0)),
          pl.BlockSpec((_H, 1), lambda i: (0, 0)),
      ],
      out_specs=pl.BlockSpec((bn, 1), lambda i: (i, 0)),
      out_shape=jax.ShapeDtypeStruct((_N, 1), _f32),
  )(x, agg_lo, agg_hi, W1, b1r, W2)


def _final_tc(s0p2, a0, a1, b2r, gum2):
  def body(s_r, a0_r, a1_r, b2_r, g_r, pred_r, act_r, lp_r):
    rows = lax.broadcasted_iota(jnp.int32, (80, 128), 0)
    cols = lax.broadcasted_iota(jnp.int32, (80, 128), 1)
    lin = rows * 128 + cols
    valid = lin < _N
    pred = s_r[...] + a0_r[...] + a1_r[...] + b2_r[0, 0]
    pred_r[...] = pred
    neg = jnp.float32(-jnp.inf)
    pm = jnp.where(valid, pred, neg)
    m = jnp.max(pm)
    e = jnp.where(valid, jnp.exp(pm - m), 0.0)
    z = jnp.sum(e)
    p = e / z
    l = jnp.log(p + 1e-20)
    v = jnp.where(valid, l + g_r[...], neg)
    m2 = jnp.max(v)
    idx = jnp.min(jnp.where(v == m2, lin, jnp.int32(2 ** 30)))
    act_r[...] = jnp.full((1, 1), idx, jnp.int32)
    psel = jnp.sum(jnp.where(lin == idx, p, 0.0))
    lp_r[...] = jnp.full((1, 1), jnp.log(psel), _f32)

  return pl.pallas_call(
      body,
      out_shape=(jax.ShapeDtypeStruct((80, 128), _f32),
                 jax.ShapeDtypeStruct((1, 1), jnp.int32),
                 jax.ShapeDtypeStruct((1, 1), _f32)),
  )(s0p2, a0, a1, b2r, gum2)


def kernel(x, edge_index, batch, W1, b1, W2, b2):
  src = edge_index[0]
  dst = edge_index[1]
  pad = _EP - _E
  padi = jnp.arange(pad, dtype=jnp.int32)
  srcp = jnp.concatenate([src, lax.rem(padi, 256)])
  dstp = jnp.concatenate([dst, _N + lax.rem(padi, _NP - _N)])
  src2d = srcp.reshape(_EP // 128, 128)
  dst2d = dstp.reshape(_EP // 128, 128)
  e3 = jnp.stack([src2d, dst2d], axis=1)

  xlo = x[:, :128]
  xhi = x[:, 128:]
  alo, ahi = _agg1_sc(xlo, xhi, e3)

  s0 = _mlp_tc(x, alo, ahi, W1, b1.reshape(1, _H), W2)
  s0p = jnp.concatenate([s0[:, 0], jnp.zeros((_NP - _N,), _f32)])
  g01 = _aggs_sc(s0p, src2d, dst2d)
  g0 = g01[0]
  g1 = g01[1]

  gum = jax.random.gumbel(jax.random.key(12345), (_N,), _f32)
  gump = jnp.concatenate([gum, jnp.zeros((_NP - _N,), _f32)])

  predp, act, lp = _final_tc(
      s0p.reshape(80, 128), g0.reshape(80, 128), g1.reshape(80, 128),
      b2.reshape(1, 1), gump.reshape(80, 128))

  pred = predp.reshape(_NP, 1)[:_N]
  action_index = act.reshape((1,)).astype(jnp.int32)
  log_prob = lp
  return pred, action_index, log_prob


# speedup vs baseline: 13.2793x; 1.2347x over previous
import functools

import jax
import jax.numpy as jnp
from jax import lax
from jax.experimental import pallas as pl
from jax.experimental.pallas import tpu as pltpu
from jax.experimental.pallas import tpu_sc as plsc

_N = 10000
_E = 160000
_D = 256
_H = 512
_NP = 10240
_EP = 163840
_NT = 16
_CH = 80
_RPT = _NP // _NT
_f32 = jnp.float32


def _agg1_sc(xlo, xhi, e3):
  mesh = plsc.VectorSubcoreMesh(core_axis_name="c", subcore_axis_name="s")

  @functools.partial(
      pl.kernel,
      mesh=mesh,
      out_type=[jax.ShapeDtypeStruct((_NP, 128), _f32),
                jax.ShapeDtypeStruct((_NP, 128), _f32)],
      scratch_types=[
          pltpu.VMEM((2, 2, 128), jnp.int32),
          pltpu.VMEM((128, 128), _f32),
          pltpu.VMEM((128, 128), _f32),
          pltpu.VMEM_SHARED((_NP, 128), _f32),
          pltpu.SemaphoreType.DMA,
          pltpu.SemaphoreType.DMA,
          pltpu.SemaphoreType.DMA,
          pltpu.SemaphoreType.DMA,
      ],
  )
  def k(xlo_h, xhi_h, e3_h, out0, out1,
        ix, b0, b1, acc, semi0, semi1, semg0, semg1):
    c = lax.axis_index("c")
    s = lax.axis_index("s")
    t0 = s * _CH

    def zb(i, carry):
      r = i // 8
      cc = lax.rem(i, 8)
      b0[r, pl.ds(cc * 16, 16)] = jnp.zeros((16,), _f32)
      return carry

    lax.fori_loop(0, 1024, zb, 0)
    rowbase = s * _RPT
    for kk in range(_RPT // 128):
      pltpu.sync_copy(b0, acc.at[pl.ds(rowbase + kk * 128, 128)])
    plsc.subcore_barrier()

    def run(x_h, out_h):
      def idxload(j, par, sem):
        return pltpu.make_async_copy(e3_h.at[t0 + j], ix.at[par], sem)

      def gather(par, buf, sem):
        return pltpu.make_async_copy(x_h.at[ix.at[par, 0]], buf, sem)

      idxload(0, 0, semi0).start()
      idxload(1, 1, semi1).start()
      idxload(0, 0, semi0).wait()
      gather(0, b0, semg0).start()
      idxload(1, 1, semi1).wait()
      gather(1, b1, semg1).start()

      def body(i, carry):
        j0 = i * 2
        more = i < _CH // 2 - 1
        gather(0, b0, semg0).wait()
        pltpu.sync_copy(b0, acc.at[ix.at[0, 1]], add=True)

        @pl.when(more)
        def _():
          idxload(j0 + 2, 0, semi0).start()

        gather(1, b1, semg1).wait()
        pltpu.sync_copy(b1, acc.at[ix.at[1, 1]], add=True)

        @pl.when(more)
        def _():
          idxload(j0 + 3, 1, semi1).start()
          idxload(0, 0, semi0).wait()
          gather(0, b0, semg0).start()
          idxload(0, 1, semi1).wait()
          gather(1, b1, semg1).start()

        return carry

      lax.fori_loop(0, _CH // 2, body, 0)
      plsc.subcore_barrier()
      pltpu.sync_copy(acc.at[pl.ds(rowbase, _RPT)],
                      out_h.at[pl.ds(rowbase, _RPT)])

    @pl.when(c == 0)
    def _():
      run(xlo_h, out0)

    @pl.when(c == 1)
    def _():
      run(xhi_h, out1)

  return k(xlo, xhi, e3)


def _aggs_sc(s0p, src2d, dst2d):
  mesh = plsc.VectorSubcoreMesh(core_axis_name="c", subcore_axis_name="s")
  ch2 = _EP // 128 // 32

  @functools.partial(
      pl.kernel,
      mesh=mesh,
      out_type=jax.ShapeDtypeStruct((2, _NP), _f32),
      scratch_types=[
          pltpu.VMEM((ch2, 128), jnp.int32),
          pltpu.VMEM((ch2, 128), jnp.int32),
          pltpu.VMEM((128,), _f32),
          pltpu.VMEM_SHARED((_NP,), _f32),
          pltpu.VMEM_SHARED((_NP,), _f32),
      ],
  )
  def k(s_h, s2_h, d2_h, out, src_v, dst_v, vals_v, s_sh, acc):
    c = lax.axis_index("c")
    s = lax.axis_index("s")
    wid = s * 2 + c
    t0 = wid * ch2
    pltpu.sync_copy(s2_h.at[pl.ds(t0, ch2)], src_v)
    pltpu.sync_copy(d2_h.at[pl.ds(t0, ch2)], dst_v)
    for g in range(8):
      vals_v[pl.ds(g * 16, 16)] = jnp.zeros((16,), _f32)
    rowbase = s * _RPT
    for kk in range(_RPT // 128):
      pltpu.sync_copy(vals_v, acc.at[pl.ds(rowbase + kk * 128, 128)])

    @pl.when(s == 0)
    def _():
      pltpu.sync_copy(s_h, s_sh)

    plsc.subcore_barrier()

    def body(j, carry):
      pltpu.sync_copy(s_sh.at[src_v.at[j]], vals_v)
      pltpu.sync_copy(vals_v, acc.at[dst_v.at[j]], add=True)
      return carry

    lax.fori_loop(0, ch2, body, 0)
    plsc.subcore_barrier()
    pltpu.sync_copy(acc.at[pl.ds(rowbase, _RPT)],
                    out.at[c, pl.ds(rowbase, _RPT)])

  return k(s0p, src2d, dst2d)


def _mlp_tc(x, agg_lo, agg_hi, W1, b1r, W2):
  bn = 1000

  def body(x_r, alo_r, ahi_r, w1_r, b1_r, w2_r, s0_r):
    hi = jnp.dot(x_r[:, :128] + alo_r[...], w1_r[:128, :],
                 preferred_element_type=_f32)
    hi = hi + jnp.dot(x_r[:, 128:] + ahi_r[...], w1_r[128:, :],
                      preferred_element_type=_f32)
    h = jnp.maximum(hi + b1_r[...], 0.0)
    s0_r[...] = jnp.dot(h, w2_r[...], preferred_element_type=_f32)

  return pl.pallas_call(
      body,
      grid=(_N // bn,),
      in_specs=[
          pl.BlockSpec((bn, _D), lambda i: (i, 0)),
          pl.BlockSpec((bn, 128), lambda i: (i, 0)),
          pl.BlockSpec((bn, 128), lambda i: (i, 0)),
          pl.BlockSpec((_D, _H), lambda i: (0, 0)),
          pl.BlockSpec((1, _H), lambda i: (0, 0)),
          pl.BlockSpec((_H, 1), lambda i: (0, 0)),
      ],
      out_specs=pl.BlockSpec((bn, 1), lambda i: (i, 0)),
      out_shape=jax.ShapeDtypeStruct((_N, 1), _f32),
  )(x, agg_lo, agg_hi, W1, b1r, W2)


def _final_tc(s0p2, a0, a1, b2r, gum2):
  def body(s_r, a0_r, a1_r, b2_r, g_r, pred_r, act_r, lp_r):
    rows = lax.broadcasted_iota(jnp.int32, (80, 128), 0)
    cols = lax.broadcasted_iota(jnp.int32, (80, 128), 1)
    lin = rows * 128 + cols
    valid = lin < _N
    pred = s_r[...] + a0_r[...] + a1_r[...] + b2_r[0, 0]
    pred_r[...] = pred
    neg = jnp.float32(-jnp.inf)
    pm = jnp.where(valid, pred, neg)
    m = jnp.max(pm)
    e = jnp.where(valid, jnp.exp(pm - m), 0.0)
    z = jnp.sum(e)
    p = e / z
    l = jnp.log(p + 1e-20)
    v = jnp.where(valid, l + g_r[...], neg)
    m2 = jnp.max(v)
    idx = jnp.min(jnp.where(v == m2, lin, jnp.int32(2 ** 30)))
    act_r[...] = jnp.full((1, 1), idx, jnp.int32)
    psel = jnp.sum(jnp.where(lin == idx, p, 0.0))
    lp_r[...] = jnp.full((1, 1), jnp.log(psel), _f32)

  return pl.pallas_call(
      body,
      out_shape=(jax.ShapeDtypeStruct((80, 128), _f32),
                 jax.ShapeDtypeStruct((1, 1), jnp.int32),
                 jax.ShapeDtypeStruct((1, 1), _f32)),
  )(s0p2, a0, a1, b2r, gum2)


def kernel(x, edge_index, batch, W1, b1, W2, b2):
  src = edge_index[0]
  dst = edge_index[1]
  pad = _EP - _E
  padi = jnp.arange(pad, dtype=jnp.int32)
  srcp = jnp.concatenate([src, lax.rem(padi, 256)])
  dstp = jnp.concatenate([dst, _N + lax.rem(padi, _NP - _N)])
  src2d = srcp.reshape(_EP // 128, 128)
  dst2d = dstp.reshape(_EP // 128, 128)
  e3 = jnp.stack([src2d, dst2d], axis=1)

  xlo = x[:, :128]
  xhi = x[:, 128:]
  alo, ahi = _agg1_sc(xlo, xhi, e3)

  s0 = _mlp_tc(x, alo, ahi, W1, b1.reshape(1, _H), W2)
  s0p = jnp.concatenate([s0[:, 0], jnp.zeros((_NP - _N,), _f32)])
  g01 = _aggs_sc(s0p, src2d, dst2d)
  g0 = g01[0]
  g1 = g01[1]

  gum = jax.random.gumbel(jax.random.key(12345), (_N,), _f32)
  gump = jnp.concatenate([gum, jnp.zeros((_NP - _N,), _f32)])

  predp, act, lp = _final_tc(
      s0p.reshape(80, 128), g0.reshape(80, 128), g1.reshape(80, 128),
      b2.reshape(1, 1), gump.reshape(80, 128))

  pred = predp.reshape(_NP, 1)[:_N]
  action_index = act.reshape((1,)).astype(jnp.int32)
  log_prob = lp
  return pred, action_index, log_prob


# speedup vs baseline: 14.0984x; 1.0617x over previous
import functools

import jax
import jax.numpy as jnp
from jax import lax
from jax.experimental import pallas as pl
from jax.experimental.pallas import tpu as pltpu
from jax.experimental.pallas import tpu_sc as plsc

_N = 10000
_E = 160000
_D = 256
_H = 512
_NP = 10240
_EP = 163840
_NT = 16
_CH = 80
_RPT = _NP // _NT
_f32 = jnp.float32


def _agg1_sc(xlo, xhi, e3):
  mesh = plsc.VectorSubcoreMesh(core_axis_name="c", subcore_axis_name="s")

  @functools.partial(
      pl.kernel,
      mesh=mesh,
      out_type=[jax.ShapeDtypeStruct((_NP, 128), _f32),
                jax.ShapeDtypeStruct((_NP, 128), _f32)],
      scratch_types=[
          pltpu.VMEM((2, 2, 128), jnp.int32),
          pltpu.VMEM((128, 128), _f32),
          pltpu.VMEM((128, 128), _f32),
          pltpu.VMEM_SHARED((_NP, 128), _f32),
          pltpu.SemaphoreType.DMA,
          pltpu.SemaphoreType.DMA,
          pltpu.SemaphoreType.DMA,
          pltpu.SemaphoreType.DMA,
          pltpu.SemaphoreType.DMA,
          pltpu.SemaphoreType.DMA,
      ],
  )
  def k(xlo_h, xhi_h, e3_h, out0, out1,
        ix, b0, b1, acc, si0, si1, sg0, sg1, ss0, ss1):
    c = lax.axis_index("c")
    s = lax.axis_index("s")
    t0 = s * _CH

    def zb(i, carry):
      r = i // 8
      cc = lax.rem(i, 8)
      b0[r, pl.ds(cc * 16, 16)] = jnp.zeros((16,), _f32)
      return carry

    lax.fori_loop(0, 1024, zb, 0)
    rowbase = s * _RPT
    for kk in range(_RPT // 128):
      pltpu.sync_copy(b0, acc.at[pl.ds(rowbase + kk * 128, 128)])
    plsc.subcore_barrier()

    def run(x_h, out_h):
      sis = (si0, si1)
      sgs = (sg0, sg1)

      def idxload(j, slot):
        return pltpu.make_async_copy(e3_h.at[t0 + j], ix.at[slot], sis[slot])

      def gather(slot, buf, par):
        return pltpu.make_async_copy(x_h.at[ix.at[slot, 0]], buf, sgs[par])

      sss = (ss0, ss1)

      def scatter(slot, buf):
        return pltpu.make_async_copy(buf, acc.at[ix.at[slot, 1]], sss[slot])

      idxload(0, 0).start()
      idxload(1, 1).start()
      idxload(0, 0).wait()
      gather(0, b0, 0).start()
      idxload(1, 1).wait()
      gather(1, b1, 1).start()

      def body(i, carry):
        j0 = i * 2
        more = i < _CH // 2 - 1
        gather(0, b0, 0).wait()
        scatter(0, b0).start(add=True)
        scatter(0, b0).wait()

        @pl.when(more)
        def _():
          idxload(j0 + 2, 0).start()

        gather(1, b1, 1).wait()
        scatter(1, b1).start(add=True)

        @pl.when(more)
        def _():
          idxload(0, 0).wait()
          gather(0, b0, 0).start()

        scatter(1, b1).wait()

        @pl.when(more)
        def _():
          idxload(j0 + 3, 1).start()
          idxload(1, 1).wait()
          gather(1, b1, 1).start()

        return carry

      lax.fori_loop(0, _CH // 2, body, 0)
      plsc.subcore_barrier()
      pltpu.sync_copy(acc.at[pl.ds(rowbase, _RPT)],
                      out_h.at[pl.ds(rowbase, _RPT)])

    @pl.when(c == 0)
    def _():
      run(xlo_h, out0)

    @pl.when(c == 1)
    def _():
      run(xhi_h, out1)

  return k(xlo, xhi, e3)


def _aggs_sc(s0p, src2d, dst2d):
  mesh = plsc.VectorSubcoreMesh(core_axis_name="c", subcore_axis_name="s")
  ch2 = _EP // 128 // 32

  @functools.partial(
      pl.kernel,
      mesh=mesh,
      out_type=jax.ShapeDtypeStruct((2, _NP), _f32),
      scratch_types=[
          pltpu.VMEM((ch2, 128), jnp.int32),
          pltpu.VMEM((ch2, 128), jnp.int32),
          pltpu.VMEM((128,), _f32),
          pltpu.VMEM_SHARED((_NP,), _f32),
          pltpu.VMEM_SHARED((_NP,), _f32),
      ],
  )
  def k(s_h, s2_h, d2_h, out, src_v, dst_v, vals_v, s_sh, acc):
    c = lax.axis_index("c")
    s = lax.axis_index("s")
    wid = s * 2 + c
    t0 = wid * ch2
    pltpu.sync_copy(s2_h.at[pl.ds(t0, ch2)], src_v)
    pltpu.sync_copy(d2_h.at[pl.ds(t0, ch2)], dst_v)
    for g in range(8):
      vals_v[pl.ds(g * 16, 16)] = jnp.zeros((16,), _f32)
    rowbase = s * _RPT
    for kk in range(_RPT // 128):
      pltpu.sync_copy(vals_v, acc.at[pl.ds(rowbase + kk * 128, 128)])

    @pl.when(s == 0)
    def _():
      pltpu.sync_copy(s_h, s_sh)

    plsc.subcore_barrier()

    def body(j, carry):
      pltpu.sync_copy(s_sh.at[src_v.at[j]], vals_v)
      pltpu.sync_copy(vals_v, acc.at[dst_v.at[j]], add=True)
      return carry

    lax.fori_loop(0, ch2, body, 0)
    plsc.subcore_barrier()
    pltpu.sync_copy(acc.at[pl.ds(rowbase, _RPT)],
                    out.at[c, pl.ds(rowbase, _RPT)])

  return k(s0p, src2d, dst2d)


def _mlp_tc(x, agg_lo, agg_hi, W1, b1r, W2):
  bn = 1000

  def body(x_r, alo_r, ahi_r, w1_r, b1_r, w2_r, s0_r):
    hi = jnp.dot(x_r[:, :128] + alo_r[...], w1_r[:128, :],
                 preferred_element_type=_f32)
    hi = hi + jnp.dot(x_r[:, 128:] + ahi_r[...], w1_r[128:, :],
                      preferred_element_type=_f32)
    h = jnp.maximum(hi + b1_r[...], 0.0)
    s0_r[...] = jnp.dot(h, w2_r[...], preferred_element_type=_f32)

  return pl.pallas_call(
      body,
      grid=(_N // bn,),
      in_specs=[
          pl.BlockSpec((bn, _D), lambda i: (i, 0)),
          pl.BlockSpec((bn, 128), lambda i: (i, 0)),
          pl.BlockSpec((bn, 128), lambda i: (i, 0)),
          pl.BlockSpec((_D, _H), lambda i: (0, 0)),
          pl.BlockSpec((1, _H), lambda i: (0, 0)),
          pl.BlockSpec((_H, 1), lambda i: (0, 0)),
      ],
      out_specs=pl.BlockSpec((bn, 1), lambda i: (i, 0)),
      out_shape=jax.ShapeDtypeStruct((_N, 1), _f32),
  )(x, agg_lo, agg_hi, W1, b1r, W2)


def _final_tc(s0p2, a0, a1, b2r, gum2):
  def body(s_r, a0_r, a1_r, b2_r, g_r, pred_r, act_r, lp_r):
    rows = lax.broadcasted_iota(jnp.int32, (80, 128), 0)
    cols = lax.broadcasted_iota(jnp.int32, (80, 128), 1)
    lin = rows * 128 + cols
    valid = lin < _N
    pred = s_r[...] + a0_r[...] + a1_r[...] + b2_r[0, 0]
    pred_r[...] = pred
    neg = jnp.float32(-jnp.inf)
    pm = jnp.where(valid, pred, neg)
    m = jnp.max(pm)
    e = jnp.where(valid, jnp.exp(pm - m), 0.0)
    z = jnp.sum(e)
    p = e / z
    l = jnp.log(p + 1e-20)
    v = jnp.where(valid, l + g_r[...], neg)
    m2 = jnp.max(v)
    idx = jnp.min(jnp.where(v == m2, lin, jnp.int32(2 ** 30)))
    act_r[...] = jnp.full((1, 1), idx, jnp.int32)
    psel = jnp.sum(jnp.where(lin == idx, p, 0.0))
    lp_r[...] = jnp.full((1, 1), jnp.log(psel), _f32)

  return pl.pallas_call(
      body,
      out_shape=(jax.ShapeDtypeStruct((80, 128), _f32),
                 jax.ShapeDtypeStruct((1, 1), jnp.int32),
                 jax.ShapeDtypeStruct((1, 1), _f32)),
  )(s0p2, a0, a1, b2r, gum2)


def kernel(x, edge_index, batch, W1, b1, W2, b2):
  src = edge_index[0]
  dst = edge_index[1]
  pad = _EP - _E
  padi = jnp.arange(pad, dtype=jnp.int32)
  srcp = jnp.concatenate([src, lax.rem(padi, 256)])
  dstp = jnp.concatenate([dst, _N + lax.rem(padi, _NP - _N)])
  src2d = srcp.reshape(_EP // 128, 128)
  dst2d = dstp.reshape(_EP // 128, 128)
  e3 = jnp.stack([src2d, dst2d], axis=1)

  xlo = x[:, :128]
  xhi = x[:, 128:]
  alo, ahi = _agg1_sc(xlo, xhi, e3)

  s0 = _mlp_tc(x, alo, ahi, W1, b1.reshape(1, _H), W2)
  s0p = jnp.concatenate([s0[:, 0], jnp.zeros((_NP - _N,), _f32)])
  g01 = _aggs_sc(s0p, src2d, dst2d)
  g0 = g01[0]
  g1 = g01[1]

  gum = jax.random.gumbel(jax.random.key(12345), (_N,), _f32)
  gump = jnp.concatenate([gum, jnp.zeros((_NP - _N,), _f32)])

  predp, act, lp = _final_tc(
      s0p.reshape(80, 128), g0.reshape(80, 128), g1.reshape(80, 128),
      b2.reshape(1, 1), gump.reshape(80, 128))

  pred = predp.reshape(_NP, 1)[:_N]
  action_index = act.reshape((1,)).astype(jnp.int32)
  log_prob = lp
  return pred, action_index, log_prob


# speedup vs baseline: 14.3067x; 1.0148x over previous
import functools

import jax
import jax.numpy as jnp
from jax import lax
from jax.experimental import pallas as pl
from jax.experimental.pallas import tpu as pltpu
from jax.experimental.pallas import tpu_sc as plsc

_N = 10000
_E = 160000
_D = 256
_H = 512
_NP = 10240
_EP = 163840
_NT = 16
_CH = 80
_RPT = _NP // _NT
_f32 = jnp.float32


def _agg1_sc(xlo, xhi, e3):
  mesh = plsc.VectorSubcoreMesh(core_axis_name="c", subcore_axis_name="s")

  @functools.partial(
      pl.kernel,
      mesh=mesh,
      out_type=[jax.ShapeDtypeStruct((_NP, 128), _f32),
                jax.ShapeDtypeStruct((_NP, 128), _f32)],
      scratch_types=[
          pltpu.VMEM((2, 2, 128), jnp.int32),
          pltpu.VMEM((128, 128), _f32),
          pltpu.VMEM((128, 128), _f32),
          pltpu.VMEM_SHARED((_NP, 128), _f32),
          pltpu.SemaphoreType.DMA,
          pltpu.SemaphoreType.DMA,
          pltpu.SemaphoreType.DMA,
          pltpu.SemaphoreType.DMA,
          pltpu.SemaphoreType.DMA,
          pltpu.SemaphoreType.DMA,
      ],
  )
  def k(xlo_h, xhi_h, e3_h, out0, out1,
        ix, b0, b1, acc, si0, si1, sg0, sg1, ss0, ss1):
    c = lax.axis_index("c")
    s = lax.axis_index("s")
    t0 = s * _CH

    def zb(i, carry):
      r = i // 8
      cc = lax.rem(i, 8)
      b0[r, pl.ds(cc * 16, 16)] = jnp.zeros((16,), _f32)
      return carry

    lax.fori_loop(0, 1024, zb, 0)
    rowbase = s * _RPT
    for kk in range(_RPT // 128):
      pltpu.sync_copy(b0, acc.at[pl.ds(rowbase + kk * 128, 128)])
    plsc.subcore_barrier()

    def run(x_h, out_h):
      sis = (si0, si1)
      sgs = (sg0, sg1)

      def idxload(j, slot):
        return pltpu.make_async_copy(e3_h.at[t0 + j], ix.at[slot], sis[slot])

      def gather(slot, buf, par):
        return pltpu.make_async_copy(x_h.at[ix.at[slot, 0]], buf, sgs[par])

      sss = (ss0, ss1)

      def scatter(slot, buf):
        return pltpu.make_async_copy(buf, acc.at[ix.at[slot, 1]], sss[slot])

      idxload(0, 0).start()
      idxload(1, 1).start()
      idxload(0, 0).wait()
      gather(0, b0, 0).start()
      idxload(1, 1).wait()
      gather(1, b1, 1).start()

      def body(i, carry):
        j0 = i * 2
        more = i < _CH // 2 - 1
        gather(0, b0, 0).wait()
        scatter(0, b0).start(add=True)
        scatter(0, b0).wait()

        @pl.when(more)
        def _():
          idxload(j0 + 2, 0).start()

        gather(1, b1, 1).wait()
        scatter(1, b1).start(add=True)

        @pl.when(more)
        def _():
          idxload(0, 0).wait()
          gather(0, b0, 0).start()

        scatter(1, b1).wait()

        @pl.when(more)
        def _():
          idxload(j0 + 3, 1).start()
          idxload(1, 1).wait()
          gather(1, b1, 1).start()

        return carry

      lax.fori_loop(0, _CH // 2, body, 0)
      plsc.subcore_barrier()
      pltpu.sync_copy(acc.at[pl.ds(rowbase, _RPT)],
                      out_h.at[pl.ds(rowbase, _RPT)])

    @pl.when(c == 0)
    def _():
      run(xlo_h, out0)

    @pl.when(c == 1)
    def _():
      run(xhi_h, out1)

  return k(xlo, xhi, e3)


def _aggs_sc(s0p, src2d, dst2d):
  mesh = plsc.VectorSubcoreMesh(core_axis_name="c", subcore_axis_name="s")
  ch2 = _EP // 128 // 32

  @functools.partial(
      pl.kernel,
      mesh=mesh,
      out_type=jax.ShapeDtypeStruct((2, _NP), _f32),
      scratch_types=[
          pltpu.VMEM((ch2, 128), jnp.int32),
          pltpu.VMEM((ch2, 128), jnp.int32),
          pltpu.VMEM((128,), _f32),
          pltpu.VMEM_SHARED((_NP,), _f32),
          pltpu.VMEM_SHARED((_NP,), _f32),
      ],
  )
  def k(s_h, s2_h, d2_h, out, src_v, dst_v, vals_v, s_sh, acc):
    c = lax.axis_index("c")
    s = lax.axis_index("s")
    wid = s * 2 + c
    t0 = wid * ch2
    pltpu.sync_copy(s2_h.at[pl.ds(t0, ch2)], src_v)
    pltpu.sync_copy(d2_h.at[pl.ds(t0, ch2)], dst_v)
    for g in range(8):
      vals_v[pl.ds(g * 16, 16)] = jnp.zeros((16,), _f32)
    rowbase = s * _RPT
    for kk in range(_RPT // 128):
      pltpu.sync_copy(vals_v, acc.at[pl.ds(rowbase + kk * 128, 128)])

    @pl.when(s == 0)
    def _():
      pltpu.sync_copy(s_h, s_sh)

    plsc.subcore_barrier()

    def body(j, carry):
      pltpu.sync_copy(s_sh.at[src_v.at[j]], vals_v)
      pltpu.sync_copy(vals_v, acc.at[dst_v.at[j]], add=True)
      return carry

    lax.fori_loop(0, ch2, body, 0)
    plsc.subcore_barrier()
    pltpu.sync_copy(acc.at[pl.ds(rowbase, _RPT)],
                    out.at[c, pl.ds(rowbase, _RPT)])

  return k(s0p, src2d, dst2d)


def _mlp_tc(x, agg_lo, agg_hi, W1, b1r, W2):
  bn = 2000

  def body(x_r, alo_r, ahi_r, w1_r, b1_r, w2_r, s0_r):
    hi = jnp.dot(x_r[:, :128] + alo_r[...], w1_r[:128, :],
                 preferred_element_type=_f32)
    hi = hi + jnp.dot(x_r[:, 128:] + ahi_r[...], w1_r[128:, :],
                      preferred_element_type=_f32)
    h = jnp.maximum(hi + b1_r[...], 0.0)
    s0_r[...] = jnp.dot(h, w2_r[...], preferred_element_type=_f32)

  return pl.pallas_call(
      body,
      grid=(_N // bn,),
      in_specs=[
          pl.BlockSpec((bn, _D), lambda i: (i, 0)),
          pl.BlockSpec((bn, 128), lambda i: (i, 0)),
          pl.BlockSpec((bn, 128), lambda i: (i, 0)),
          pl.BlockSpec((_D, _H), lambda i: (0, 0)),
          pl.BlockSpec((1, _H), lambda i: (0, 0)),
          pl.BlockSpec((_H, 1), lambda i: (0, 0)),
      ],
      out_specs=pl.BlockSpec((bn, 1), lambda i: (i, 0)),
      out_shape=jax.ShapeDtypeStruct((_N, 1), _f32),
  )(x, agg_lo, agg_hi, W1, b1r, W2)


def _final_tc(s0p2, a0, a1, b2r, gum2):
  def body(s_r, a0_r, a1_r, b2_r, g_r, pred_r, act_r, lp_r):
    rows = lax.broadcasted_iota(jnp.int32, (80, 128), 0)
    cols = lax.broadcasted_iota(jnp.int32, (80, 128), 1)
    lin = rows * 128 + cols
    valid = lin < _N
    pred = s_r[...] + a0_r[...] + a1_r[...] + b2_r[0, 0]
    pred_r[...] = pred
    neg = jnp.float32(-jnp.inf)
    pm = jnp.where(valid, pred, neg)
    m = jnp.max(pm)
    e = jnp.where(valid, jnp.exp(pm - m), 0.0)
    z = jnp.sum(e)
    p = e / z
    l = jnp.log(p + 1e-20)
    v = jnp.where(valid, l + g_r[...], neg)
    m2 = jnp.max(v)
    idx = jnp.min(jnp.where(v == m2, lin, jnp.int32(2 ** 30)))
    act_r[...] = jnp.full((1, 1), idx, jnp.int32)
    psel = jnp.sum(jnp.where(lin == idx, p, 0.0))
    lp_r[...] = jnp.full((1, 1), jnp.log(psel), _f32)

  return pl.pallas_call(
      body,
      out_shape=(jax.ShapeDtypeStruct((80, 128), _f32),
                 jax.ShapeDtypeStruct((1, 1), jnp.int32),
                 jax.ShapeDtypeStruct((1, 1), _f32)),
  )(s0p2, a0, a1, b2r, gum2)


def kernel(x, edge_index, batch, W1, b1, W2, b2):
  src = edge_index[0]
  dst = edge_index[1]
  pad = _EP - _E
  padi = jnp.arange(pad, dtype=jnp.int32)
  srcp = jnp.concatenate([src, lax.rem(padi, 256)])
  dstp = jnp.concatenate([dst, _N + lax.rem(padi, _NP - _N)])
  src2d = srcp.reshape(_EP // 128, 128)
  dst2d = dstp.reshape(_EP // 128, 128)
  e3 = jnp.stack([src2d, dst2d], axis=1)

  xlo = x[:, :128]
  xhi = x[:, 128:]
  alo, ahi = _agg1_sc(xlo, xhi, e3)

  s0 = _mlp_tc(x, alo, ahi, W1, b1.reshape(1, _H), W2)
  s0p = jnp.concatenate([s0[:, 0], jnp.zeros((_NP - _N,), _f32)])
  g01 = _aggs_sc(s0p, src2d, dst2d)
  g0 = g01[0]
  g1 = g01[1]

  gum = jax.random.gumbel(jax.random.key(12345), (_N,), _f32)
  gump = jnp.concatenate([gum, jnp.zeros((_NP - _N,), _f32)])

  predp, act, lp = _final_tc(
      s0p.reshape(80, 128), g0.reshape(80, 128), g1.reshape(80, 128),
      b2.reshape(1, 1), gump.reshape(80, 128))

  pred = predp.reshape(_NP, 1)[:_N]
  action_index = act.reshape((1,)).astype(jnp.int32)
  log_prob = lp
  return pred, action_index, log_prob
